# baseline (speedup 1.0000x reference)
import functools
import math

import jax
import jax.numpy as jnp
from jax import lax
from jax.experimental import pallas as pl
from jax.experimental.pallas import tpu as pltpu
from jax.experimental.pallas import tpu_sc as plsc

B, S, D, H, C, NB, DFF, V = 2, 4096, 1024, 8, 128, 32, 4096, 256
DH = D // H
NC = S // C
BH = B * H
NBK = 2 * NB


_SB1 = 512


def _stage1_body(x_ref, g_ref, b_ref, wqk_ref, wv_ref, rot_ref,
                 qk_ref, v_ref, bk_ref):
    x = x_ref[0]
    mu = jnp.mean(x, axis=-1, keepdims=True)
    var = jnp.mean((x - mu) * (x - mu), axis=-1, keepdims=True)
    h = (x - mu) * lax.rsqrt(var + 1e-6) * g_ref[...] + b_ref[...]
    qk = jnp.dot(h, wqk_ref[...], preferred_element_type=jnp.float32)
    v = jnp.dot(h.astype(jnp.bfloat16), wv_ref[...],
                preferred_element_type=jnp.float32)
    rx = jnp.dot(qk, rot_ref[...], preferred_element_type=jnp.float32)
    cols = []
    for hh in range(H):
        qk_ref[0, hh] = qk[:, hh * DH:(hh + 1) * DH]
        v_ref[0, hh] = v[:, hh * DH:(hh + 1) * DH]
        r = rx[:, hh * NBK:(hh + 1) * NBK]
        m = jnp.max(r, axis=-1, keepdims=True)
        ii = lax.broadcasted_iota(jnp.int32, r.shape, 1)
        idx = jnp.min(jnp.where(r == m, ii, NBK), axis=-1)
        cols.append(idx[:, None])
    bk_ref[0] = jnp.concatenate(cols, axis=1)


def _stage1(x, ln1_g, ln1_b, Wqk, Wv, rot_bd, interpret=False):
    grid = (B, S // _SB1)
    return pl.pallas_call(
        _stage1_body,
        grid=grid,
        in_specs=[
            pl.BlockSpec((1, _SB1, D), lambda b, i: (b, i, 0)),
            pl.BlockSpec((D,), lambda b, i: (0,)),
            pl.BlockSpec((D,), lambda b, i: (0,)),
            pl.BlockSpec((D, D), lambda b, i: (0, 0)),
            pl.BlockSpec((D, D), lambda b, i: (0, 0)),
            pl.BlockSpec((D, H * NBK), lambda b, i: (0, 0)),
        ],
        out_specs=[
            pl.BlockSpec((1, H, _SB1, DH), lambda b, i: (b, 0, i, 0)),
            pl.BlockSpec((1, H, _SB1, DH), lambda b, i: (b, 0, i, 0)),
            pl.BlockSpec((1, _SB1, H), lambda b, i: (b, i, 0)),
        ],
        out_shape=[
            jax.ShapeDtypeStruct((B, H, S, DH), jnp.float32),
            jax.ShapeDtypeStruct((B, H, S, DH), jnp.float32),
            jax.ShapeDtypeStruct((B, S, H), jnp.int32),
        ],
        interpret=interpret,
    )(x, ln1_g, ln1_b, Wqk, Wv, rot_bd)


def _attn_body(q_ref, v_ref, t_ref, o_ref):
    p = pl.program_id(0)
    qk_all = q_ref[0]
    nrm = jnp.sqrt(jnp.sum(qk_all * qk_all, axis=-1, keepdims=True))
    k_all = (qk_all / (nrm + 1e-6)).astype(jnp.bfloat16)
    q_allb = qk_all.astype(jnp.bfloat16)
    v_allb = v_ref[0].astype(jnp.bfloat16)
    t_all = (t_ref[0] - p * S).astype(jnp.float32)
    k_ext = jnp.concatenate([k_all[S - C:], k_all], axis=0)
    v_ext = jnp.concatenate([v_allb[S - C:], v_allb], axis=0)
    t_ext = jnp.concatenate([t_all[:, S - C:], t_all], axis=1)
    scale = 1.0 / math.sqrt(DH)
    for n in range(NC):
        q = q_allb[n * C:(n + 1) * C]
        k2 = k_ext[n * C:(n + 2) * C]
        v2 = v_ext[n * C:(n + 2) * C]
        s = lax.dot_general(q, k2, (((1,), (1,)), ((), ())),
                            preferred_element_type=jnp.float32) * scale
        tq = t_all[:, n * C:(n + 1) * C]
        tk = t_ext[:, n * C:(n + 2) * C]
        tqc = jnp.transpose(tq)
        s = jnp.where(tqc < tk, -1e9, s)
        s = jnp.where(tqc == tk, -1e5, s)
        m = jnp.max(s, axis=-1, keepdims=True)
        e = jnp.exp(s - m)
        a = (e / jnp.sum(e, axis=-1, keepdims=True)).astype(jnp.bfloat16)
        o_ref[0, n * C:(n + 1) * C, :] = jnp.dot(
            a, v2, preferred_element_type=jnp.float32)


def _attention(qks, vs, ts, interpret=False):
    grid = (BH,)
    return pl.pallas_call(
        _attn_body,
        grid=grid,
        in_specs=[
            pl.BlockSpec((1, S, DH), lambda p: (p, 0, 0)),
            pl.BlockSpec((1, S, DH), lambda p: (p, 0, 0)),
            pl.BlockSpec((1, 1, S), lambda p: (p, 0, 0)),
        ],
        out_specs=pl.BlockSpec((1, S, DH), lambda p: (p, 0, 0)),
        out_shape=jax.ShapeDtypeStruct((BH, S, DH), jnp.float32),
        interpret=interpret,
    )(qks, vs, ts)


_SB6 = 256
_KCH = 4


def _stage6_body(x_ref, o_ref, wo_ref, g2_ref, b2_ref, w1_ref, bb1_ref,
                 w2_ref, bb2_ref, wout_ref, bout_ref, out_ref):
    R = 2 * _SB6
    bf = jnp.bfloat16
    x2 = x_ref[...].reshape(R, D)
    acc_o = jnp.zeros((R, D), jnp.float32)
    for hh in range(H):
        o_h = o_ref[:, hh].reshape(R, DH).astype(bf)
        acc_o = acc_o + jnp.dot(o_h, wo_ref[hh * DH:(hh + 1) * DH],
                                preferred_element_type=jnp.float32)
    y1 = x2 + acc_o
    mu = jnp.mean(y1, axis=-1, keepdims=True)
    var = jnp.mean((y1 - mu) * (y1 - mu), axis=-1, keepdims=True)
    h2 = ((y1 - mu) * lax.rsqrt(var + 1e-6) * g2_ref[...]
          + b2_ref[...]).astype(bf)
    kc = DFF // _KCH
    acc = jnp.zeros((R, D), jnp.float32)
    for j in range(_KCH):
        hj = jnp.dot(h2, w1_ref[:, j * kc:(j + 1) * kc],
                     preferred_element_type=jnp.float32)
        hj = jnp.maximum(hj + bb1_ref[j * kc:(j + 1) * kc], 0.0)
        acc = acc + jnp.dot(hj.astype(bf), w2_ref[j * kc:(j + 1) * kc, :],
                            preferred_element_type=jnp.float32)
    y2 = x2 + acc + bb2_ref[...]
    lg = (jnp.dot(y1.astype(bf), wout_ref[:D],
                  preferred_element_type=jnp.float32)
          + jnp.dot(y2.astype(bf), wout_ref[D:],
                    preferred_element_type=jnp.float32)
          + bout_ref[...])
    l0 = lg[:_SB6]
    l1 = lg[_SB6:]
    m = jnp.maximum(l0, l1)
    e0 = jnp.exp(l0 - m)
    e1 = jnp.exp(l1 - m)
    ssum = e0 + e1
    out_ref[0] = e0 / ssum
    out_ref[1] = e1 / ssum


def _stage6(x, o_t, Wo, ln2_g, ln2_b, W1, b1, W2, b2, Wout, bout,
            interpret=False):
    grid = (S // _SB6,)
    return pl.pallas_call(
        _stage6_body,
        grid=grid,
        in_specs=[
            pl.BlockSpec((B, _SB6, D), lambda i: (0, i, 0)),
            pl.BlockSpec((B, H, _SB6, DH), lambda i: (0, 0, i, 0)),
            pl.BlockSpec((D, D), lambda i: (0, 0)),
            pl.BlockSpec((D,), lambda i: (0,)),
            pl.BlockSpec((D,), lambda i: (0,)),
            pl.BlockSpec((D, DFF), lambda i: (0, 0)),
            pl.BlockSpec((DFF,), lambda i: (0,)),
            pl.BlockSpec((DFF, D), lambda i: (0, 0)),
            pl.BlockSpec((D,), lambda i: (0,)),
            pl.BlockSpec((2 * D, V), lambda i: (0, 0)),
            pl.BlockSpec((V,), lambda i: (0,)),
        ],
        out_specs=pl.BlockSpec((B, _SB6, V), lambda i: (0, i, 0)),
        out_shape=jax.ShapeDtypeStruct((B, S, V), jnp.float32),
        interpret=interpret,
    )(x, o_t, Wo, ln2_g, ln2_b, W1, b1, W2, b2, Wout, bout)


_SEG = S // 16
_HW = NBK * BH


def _sc_sort_gather(bk_flat, qk2, v2):
    mesh = plsc.VectorSubcoreMesh(core_axis_name="c", subcore_axis_name="s")

    @functools.partial(
        pl.kernel,
        mesh=mesh,
        out_type=[jax.ShapeDtypeStruct((BH * S,), jnp.int32),
                  jax.ShapeDtypeStruct((BH, S), jnp.int32),
                  jax.ShapeDtypeStruct((BH * S, DH), jnp.float32),
                  jax.ShapeDtypeStruct((BH * S, DH), jnp.float32)],
        scratch_types=[
            pltpu.VMEM((_SEG * BH,), jnp.int32),
            pltpu.VMEM((_HW,), jnp.int32),
            pltpu.VMEM((16 * _HW,), jnp.int32),
            pltpu.VMEM((_SEG * BH // 128, 128), jnp.int32),
            pltpu.VMEM((_SEG * BH // 128, 128), jnp.int32),
            pltpu.VMEM((BH, _SEG), jnp.int32),
            pltpu.VMEM((128, DH), jnp.float32),
            pltpu.VMEM_SHARED((16, _HW), jnp.int32),
            pltpu.VMEM_SHARED((BH * S,), jnp.int32),
            pltpu.SemaphoreType.DMA,
        ],
        compiler_params=pltpu.CompilerParams(needs_layout_passes=False),
    )
    def sort_k(bk_hbm, qk_hbm, v_hbm, sidx_hbm, uidx_hbm, qks_hbm, vs_hbm,
               buf, hist, shv, idxb, valb, ubuf, rowb, shared, sres, sem):
        cid = lax.axis_index("c")
        tau = lax.axis_index("s")
        lanes = lax.broadcasted_iota(jnp.int32, (BH,), 0)

        pltpu.sync_copy(bk_hbm.at[pl.ds(tau * _SEG * BH, _SEG * BH)], buf)
        for j in range(NBK):
            hist[pl.ds(j * BH, BH)] = jnp.zeros((BH,), jnp.int32)

        def p1(i, carry):
            bk = buf[pl.ds(i * BH, BH)]
            a = bk * BH + lanes
            cnt = plsc.load_gather(hist, [a])
            plsc.store_scatter(hist, [a], cnt + 1)
            return carry

        lax.fori_loop(0, _SEG, p1, 0)

        pltpu.sync_copy(hist, shared.at[tau])
        plsc.subcore_barrier()
        for tp in range(16):
            pltpu.sync_copy(shared.at[tp], shv.at[pl.ds(tp * _HW, _HW)])

        def pj(j, g):
            tot = jnp.zeros((BH,), jnp.int32)
            part = jnp.zeros((BH,), jnp.int32)
            for tp in range(16):
                hv = shv[pl.ds(tp * _HW + j * BH, BH)]
                tot = tot + hv
                part = jnp.where(tp < tau, part + hv, part)
            hist[pl.ds(j * BH, BH)] = g + part
            return g + tot

        lax.fori_loop(0, NBK, pj, jnp.zeros((BH,), jnp.int32))

        def p3(i, carry):
            bk = buf[pl.ds(i * BH, BH)]
            a = bk * BH + lanes
            pos = plsc.load_gather(hist, [a])
            plsc.store_scatter(hist, [a], pos + 1)
            fa = i * BH + lanes
            plsc.store_scatter(idxb, [fa >> 7, fa & 127], lanes * S + pos)
            plsc.store_scatter(valb, [fa >> 7, fa & 127],
                               lanes * S + (tau * _SEG + i))
            plsc.store_scatter(ubuf, [lanes, jnp.full((BH,), i, jnp.int32)],
                               lanes * S + pos)
            return carry

        lax.fori_loop(0, _SEG, p3, 0)
        cps = [pltpu.async_copy(valb.at[j], sres.at[idxb.at[j]], sem)
               for j in range(_SEG * BH // 128)]
        for cp in cps:
            cp.wait()
        plsc.subcore_barrier()

        @pl.when(cid == 0)
        def _():
            pltpu.sync_copy(sres.at[pl.ds(tau * _SEG * BH, _SEG * BH)],
                            sidx_hbm.at[pl.ds(tau * _SEG * BH, _SEG * BH)])

        @pl.when(cid == 1)
        def _():
            pltpu.sync_copy(ubuf, uidx_hbm.at[:, pl.ds(tau * _SEG, _SEG)])

        for c in range(_SEG * BH // 128):
            pltpu.sync_copy(sres.at[pl.ds(tau * _SEG * BH + c * 128, 128)],
                            idxb.at[c])

        @pl.when(cid == 0)
        def _():
            for c in range(_SEG * BH // 128):
                pltpu.async_copy(qk_hbm.at[idxb.at[c]], rowb, sem).wait()
                pltpu.sync_copy(
                    rowb,
                    qks_hbm.at[pl.ds(tau * _SEG * BH + c * 128, 128)])

        @pl.when(cid == 1)
        def _():
            for c in range(_SEG * BH // 128):
                pltpu.async_copy(v_hbm.at[idxb.at[c]], rowb, sem).wait()
                pltpu.sync_copy(
                    rowb,
                    vs_hbm.at[pl.ds(tau * _SEG * BH + c * 128, 128)])

    return sort_k(bk_flat, qk2, v2)


_NW = 32
_RPW = BH * S // _NW
_GCH = _RPW // 128


def _sc_gather_qkv(qk2, v2, sidx2):
    mesh = plsc.VectorSubcoreMesh(core_axis_name="c", subcore_axis_name="s")

    @functools.partial(
        pl.kernel,
        mesh=mesh,
        out_type=[jax.ShapeDtypeStruct((BH * S, DH), jnp.float32),
                  jax.ShapeDtypeStruct((BH * S, DH), jnp.float32)],
        scratch_types=[
            pltpu.VMEM((_GCH, 128), jnp.int32),
            pltpu.VMEM((128, DH), jnp.float32),
            pltpu.VMEM((128, DH), jnp.float32),
            pltpu.SemaphoreType.DMA,
            pltpu.SemaphoreType.DMA,
        ],
    )
    def gather_k(qk_hbm, v_hbm, idx_hbm, qks_hbm, vs_hbm,
                 idx_v, qbuf, vbuf, sem1, sem2):
        cid = lax.axis_index("c")
        sid = lax.axis_index("s")
        wid = sid * 2 + cid
        pltpu.sync_copy(idx_hbm.at[pl.ds(wid * _GCH, _GCH)], idx_v)
        for ch in range(_GCH):
            cp1 = pltpu.async_copy(qk_hbm.at[idx_v.at[ch]], qbuf, sem1)
            cp2 = pltpu.async_copy(v_hbm.at[idx_v.at[ch]], vbuf, sem2)
            cp1.wait()
            cp2.wait()
            row0 = wid * _RPW + ch * 128
            pltpu.sync_copy(qbuf, qks_hbm.at[pl.ds(row0, 128)])
            pltpu.sync_copy(vbuf, vs_hbm.at[pl.ds(row0, 128)])

    return gather_k(qk2, v2, sidx2)


def _sc_unsort_o(o_s2, uidx2):
    mesh = plsc.VectorSubcoreMesh(core_axis_name="c", subcore_axis_name="s")

    @functools.partial(
        pl.kernel,
        mesh=mesh,
        out_type=jax.ShapeDtypeStruct((BH * S, DH), jnp.float32),
        scratch_types=[
            pltpu.VMEM((_GCH, 128), jnp.int32),
            pltpu.VMEM((128, DH), jnp.float32),
            pltpu.SemaphoreType.DMA,
        ],
    )
    def unsort_k(os_hbm, idx_hbm, ot_hbm, idx_v, buf, sem):
        cid = lax.axis_index("c")
        sid = lax.axis_index("s")
        wid = sid * 2 + cid
        pltpu.sync_copy(idx_hbm.at[pl.ds(wid * _GCH, _GCH)], idx_v)
        for ch in range(_GCH):
            pltpu.async_copy(os_hbm.at[idx_v.at[ch]], buf, sem).wait()
            pltpu.sync_copy(buf, ot_hbm.at[pl.ds(wid * _RPW + ch * 128,
                                                 128)])

    return unsort_k(o_s2, uidx2)


def _pipeline(x, rot, Wqk, Wv, Wo, ln1_g, ln1_b, W1, b1, W2, b2,
              ln2_g, ln2_b, Wout, bout, interpret=False):
    rotc = jnp.concatenate([rot, -rot], axis=-1)
    rot_bd = jnp.zeros((D, H * NBK), jnp.float32)
    for hh in range(H):
        rot_bd = rot_bd.at[hh * DH:(hh + 1) * DH,
                           hh * NBK:(hh + 1) * NBK].set(rotc[hh])

    bf = jnp.bfloat16
    Wv, Wo, W1, W2, Wout = (w.astype(bf) for w in (Wv, Wo, W1, W2, Wout))

    qk, v, bk_bsh = _stage1(x, ln1_g, ln1_b, Wqk, Wv, rot_bd,
                            interpret=interpret)

    bk_flat = bk_bsh.transpose(1, 0, 2).reshape(S * BH)
    qk2 = qk.reshape(BH * S, DH)
    v2 = v.reshape(BH * S, DH)
    s_flat, u_flat, qks, vs = _sc_sort_gather(bk_flat, qk2, v2)

    ts = s_flat.reshape(BH, 1, S)
    o_s = _attention(qks.reshape(BH, S, DH), vs.reshape(BH, S, DH), ts,
                     interpret=interpret)

    o_t = _sc_unsort_o(o_s.reshape(BH * S, DH),
                       u_flat.reshape(BH * S).reshape(-1, 128)
                       ).reshape(B, H, S, DH)

    return _stage6(x, o_t, Wo, ln2_g, ln2_b, W1, b1, W2, b2, Wout, bout,
                   interpret=interpret)


def kernel(x, rot, Wqk, Wv, Wo, ln1_g, ln1_b, W1, b1, W2, b2,
           ln2_g, ln2_b, Wout, bout):
    return _pipeline(x, rot, Wqk, Wv, Wo, ln1_g, ln1_b, W1, b1, W2, b2,
                     ln2_g, ln2_b, Wout, bout)


# speedup vs baseline: 1.0266x; 1.0266x over previous
import functools
import math

import jax
import jax.numpy as jnp
from jax import lax
from jax.experimental import pallas as pl
from jax.experimental.pallas import tpu as pltpu
from jax.experimental.pallas import tpu_sc as plsc

B, S, D, H, C, NB, DFF, V = 2, 4096, 1024, 8, 128, 32, 4096, 256
DH = D // H
NC = S // C
BH = B * H
NBK = 2 * NB


_SB1 = 512


def _stage1_body(x_ref, g_ref, b_ref, wqk_ref, wv_ref, rot_ref,
                 qk_ref, v_ref, bk_ref):
    x = x_ref[0]
    mu = jnp.mean(x, axis=-1, keepdims=True)
    var = jnp.mean((x - mu) * (x - mu), axis=-1, keepdims=True)
    h = (x - mu) * lax.rsqrt(var + 1e-6) * g_ref[...] + b_ref[...]
    qk = jnp.dot(h, wqk_ref[...], preferred_element_type=jnp.float32)
    v = jnp.dot(h.astype(jnp.bfloat16), wv_ref[...],
                preferred_element_type=jnp.float32)
    rx = jnp.dot(qk, rot_ref[...], preferred_element_type=jnp.float32)
    cols = []
    for hh in range(H):
        qk_ref[0, hh] = qk[:, hh * DH:(hh + 1) * DH]
        v_ref[0, hh] = v[:, hh * DH:(hh + 1) * DH]
        r = rx[:, hh * NBK:(hh + 1) * NBK]
        m = jnp.max(r, axis=-1, keepdims=True)
        ii = lax.broadcasted_iota(jnp.int32, r.shape, 1)
        idx = jnp.min(jnp.where(r == m, ii, NBK), axis=-1)
        cols.append(idx[:, None])
    bk_ref[0] = jnp.concatenate(cols, axis=1)


def _stage1(x, ln1_g, ln1_b, Wqk, Wv, rot_bd, interpret=False):
    grid = (B, S // _SB1)
    return pl.pallas_call(
        _stage1_body,
        grid=grid,
        in_specs=[
            pl.BlockSpec((1, _SB1, D), lambda b, i: (b, i, 0)),
            pl.BlockSpec((D,), lambda b, i: (0,)),
            pl.BlockSpec((D,), lambda b, i: (0,)),
            pl.BlockSpec((D, D), lambda b, i: (0, 0)),
            pl.BlockSpec((D, D), lambda b, i: (0, 0)),
            pl.BlockSpec((D, H * NBK), lambda b, i: (0, 0)),
        ],
        out_specs=[
            pl.BlockSpec((1, H, _SB1, DH), lambda b, i: (b, 0, i, 0)),
            pl.BlockSpec((1, H, _SB1, DH), lambda b, i: (b, 0, i, 0)),
            pl.BlockSpec((1, _SB1, H), lambda b, i: (b, i, 0)),
        ],
        out_shape=[
            jax.ShapeDtypeStruct((B, H, S, DH), jnp.float32),
            jax.ShapeDtypeStruct((B, H, S, DH), jnp.float32),
            jax.ShapeDtypeStruct((B, S, H), jnp.int32),
        ],
        interpret=interpret,
    )(x, ln1_g, ln1_b, Wqk, Wv, rot_bd)


def _attn_body(q_ref, v_ref, t_ref, o_ref):
    p = pl.program_id(0)
    qk_all = q_ref[0]
    nrm = jnp.sqrt(jnp.sum(qk_all * qk_all, axis=-1, keepdims=True))
    k_all = (qk_all / (nrm + 1e-6)).astype(jnp.bfloat16)
    q_allb = qk_all.astype(jnp.bfloat16)
    v_allb = v_ref[0].astype(jnp.bfloat16)
    t_all = (t_ref[0] - p * S).astype(jnp.float32)
    k_ext = jnp.concatenate([k_all[S - C:], k_all], axis=0)
    v_ext = jnp.concatenate([v_allb[S - C:], v_allb], axis=0)
    t_ext = jnp.concatenate([t_all[:, S - C:], t_all], axis=1)
    scale = 1.0 / math.sqrt(DH)
    for n in range(NC):
        q = q_allb[n * C:(n + 1) * C]
        k2 = k_ext[n * C:(n + 2) * C]
        v2 = v_ext[n * C:(n + 2) * C]
        s = lax.dot_general(q, k2, (((1,), (1,)), ((), ())),
                            preferred_element_type=jnp.float32) * scale
        tq = t_all[:, n * C:(n + 1) * C]
        tk = t_ext[:, n * C:(n + 2) * C]
        tqc = jnp.transpose(tq)
        s = jnp.where(tqc < tk, -1e9, s)
        s = jnp.where(tqc == tk, -1e5, s)
        m = jnp.max(s, axis=-1, keepdims=True)
        e = jnp.exp(s - m)
        a = (e / jnp.sum(e, axis=-1, keepdims=True)).astype(jnp.bfloat16)
        o_ref[0, n * C:(n + 1) * C, :] = jnp.dot(
            a, v2, preferred_element_type=jnp.float32)


def _attention(qks, vs, ts, interpret=False):
    grid = (BH,)
    return pl.pallas_call(
        _attn_body,
        grid=grid,
        in_specs=[
            pl.BlockSpec((1, S, DH), lambda p: (p, 0, 0)),
            pl.BlockSpec((1, S, DH), lambda p: (p, 0, 0)),
            pl.BlockSpec((1, 1, S), lambda p: (p, 0, 0)),
        ],
        out_specs=pl.BlockSpec((1, S, DH), lambda p: (p, 0, 0)),
        out_shape=jax.ShapeDtypeStruct((BH, S, DH), jnp.float32),
        interpret=interpret,
    )(qks, vs, ts)


_SB6 = 256
_KCH = 4


def _stage6_body(x_ref, o_ref, wo_ref, g2_ref, b2_ref, w1_ref, bb1_ref,
                 w2_ref, bb2_ref, wout_ref, bout_ref, out_ref):
    R = 2 * _SB6
    bf = jnp.bfloat16
    x2 = x_ref[...].reshape(R, D)
    acc_o = jnp.zeros((R, D), jnp.float32)
    for hh in range(H):
        o_h = o_ref[:, hh].reshape(R, DH).astype(bf)
        acc_o = acc_o + jnp.dot(o_h, wo_ref[hh * DH:(hh + 1) * DH],
                                preferred_element_type=jnp.float32)
    y1 = x2 + acc_o
    mu = jnp.mean(y1, axis=-1, keepdims=True)
    var = jnp.mean((y1 - mu) * (y1 - mu), axis=-1, keepdims=True)
    h2 = ((y1 - mu) * lax.rsqrt(var + 1e-6) * g2_ref[...]
          + b2_ref[...]).astype(bf)
    kc = DFF // _KCH
    acc = jnp.zeros((R, D), jnp.float32)
    for j in range(_KCH):
        hj = jnp.dot(h2, w1_ref[:, j * kc:(j + 1) * kc],
                     preferred_element_type=jnp.float32)
        hj = jnp.maximum(hj + bb1_ref[j * kc:(j + 1) * kc], 0.0)
        acc = acc + jnp.dot(hj.astype(bf), w2_ref[j * kc:(j + 1) * kc, :],
                            preferred_element_type=jnp.float32)
    y2 = x2 + acc + bb2_ref[...]
    lg = (jnp.dot(y1.astype(bf), wout_ref[:D],
                  preferred_element_type=jnp.float32)
          + jnp.dot(y2.astype(bf), wout_ref[D:],
                    preferred_element_type=jnp.float32)
          + bout_ref[...])
    l0 = lg[:_SB6]
    l1 = lg[_SB6:]
    m = jnp.maximum(l0, l1)
    e0 = jnp.exp(l0 - m)
    e1 = jnp.exp(l1 - m)
    ssum = e0 + e1
    out_ref[0] = e0 / ssum
    out_ref[1] = e1 / ssum


def _stage6(x, o_t, Wo, ln2_g, ln2_b, W1, b1, W2, b2, Wout, bout,
            interpret=False):
    grid = (S // _SB6,)
    return pl.pallas_call(
        _stage6_body,
        grid=grid,
        in_specs=[
            pl.BlockSpec((B, _SB6, D), lambda i: (0, i, 0)),
            pl.BlockSpec((B, H, _SB6, DH), lambda i: (0, 0, i, 0)),
            pl.BlockSpec((D, D), lambda i: (0, 0)),
            pl.BlockSpec((D,), lambda i: (0,)),
            pl.BlockSpec((D,), lambda i: (0,)),
            pl.BlockSpec((D, DFF), lambda i: (0, 0)),
            pl.BlockSpec((DFF,), lambda i: (0,)),
            pl.BlockSpec((DFF, D), lambda i: (0, 0)),
            pl.BlockSpec((D,), lambda i: (0,)),
            pl.BlockSpec((2 * D, V), lambda i: (0, 0)),
            pl.BlockSpec((V,), lambda i: (0,)),
        ],
        out_specs=pl.BlockSpec((B, _SB6, V), lambda i: (0, i, 0)),
        out_shape=jax.ShapeDtypeStruct((B, S, V), jnp.float32),
        interpret=interpret,
    )(x, o_t, Wo, ln2_g, ln2_b, W1, b1, W2, b2, Wout, bout)


_SEG = S // 16
_HW = NBK * BH


def _sc_sort(bk_flat):
    mesh = plsc.VectorSubcoreMesh(core_axis_name="c", subcore_axis_name="s")

    @functools.partial(
        pl.kernel,
        mesh=mesh,
        out_type=[jax.ShapeDtypeStruct((BH * S,), jnp.int32),
                  jax.ShapeDtypeStruct((BH, S), jnp.int32)],
        scratch_types=[
            pltpu.VMEM((_SEG * BH,), jnp.int32),
            pltpu.VMEM((_HW,), jnp.int32),
            pltpu.VMEM((16 * _HW,), jnp.int32),
            pltpu.VMEM((_SEG * BH // 128, 128), jnp.int32),
            pltpu.VMEM((_SEG * BH // 128, 128), jnp.int32),
            pltpu.VMEM((BH, _SEG), jnp.int32),
            pltpu.VMEM_SHARED((16, _HW), jnp.int32),
            pltpu.VMEM_SHARED((BH * S,), jnp.int32),
            pltpu.SemaphoreType.DMA,
        ],
        compiler_params=pltpu.CompilerParams(needs_layout_passes=False),
    )
    def sort_k(bk_hbm, sidx_hbm, uidx_hbm, buf, hist, shv, idxb, valb,
               ubuf, shared, sres, sem):
        cid = lax.axis_index("c")
        tau = lax.axis_index("s")
        lanes = lax.broadcasted_iota(jnp.int32, (BH,), 0)

        pltpu.sync_copy(bk_hbm.at[pl.ds(tau * _SEG * BH, _SEG * BH)], buf)
        for j in range(NBK):
            hist[pl.ds(j * BH, BH)] = jnp.zeros((BH,), jnp.int32)

        def p1(i, carry):
            bk = buf[pl.ds(i * BH, BH)]
            a = bk * BH + lanes
            cnt = plsc.load_gather(hist, [a])
            plsc.store_scatter(hist, [a], cnt + 1)
            return carry

        lax.fori_loop(0, _SEG, p1, 0)

        pltpu.sync_copy(hist, shared.at[tau])
        plsc.subcore_barrier()
        for tp in range(16):
            pltpu.sync_copy(shared.at[tp], shv.at[pl.ds(tp * _HW, _HW)])

        def pj(j, g):
            tot = jnp.zeros((BH,), jnp.int32)
            part = jnp.zeros((BH,), jnp.int32)
            for tp in range(16):
                hv = shv[pl.ds(tp * _HW + j * BH, BH)]
                tot = tot + hv
                part = jnp.where(tp < tau, part + hv, part)
            hist[pl.ds(j * BH, BH)] = g + part
            return g + tot

        lax.fori_loop(0, NBK, pj, jnp.zeros((BH,), jnp.int32))

        @pl.when(cid == 0)
        def _():
            def p3(i, carry):
                bk = buf[pl.ds(i * BH, BH)]
                a = bk * BH + lanes
                pos = plsc.load_gather(hist, [a])
                plsc.store_scatter(hist, [a], pos + 1)
                fa = i * BH + lanes
                plsc.store_scatter(idxb, [fa >> 7, fa & 127],
                                   lanes * S + pos)
                plsc.store_scatter(valb, [fa >> 7, fa & 127],
                                   lanes * S + (tau * _SEG + i))
                return carry

            lax.fori_loop(0, _SEG, p3, 0)
            cps = [pltpu.async_copy(valb.at[j], sres.at[idxb.at[j]], sem)
                   for j in range(_SEG * BH // 128)]
            for cp in cps:
                cp.wait()
            plsc.subcore_barrier()
            pltpu.sync_copy(sres.at[pl.ds(tau * _SEG * BH, _SEG * BH)],
                            sidx_hbm.at[pl.ds(tau * _SEG * BH, _SEG * BH)])

        @pl.when(cid == 1)
        def _():
            def p3u(i, carry):
                bk = buf[pl.ds(i * BH, BH)]
                a = bk * BH + lanes
                pos = plsc.load_gather(hist, [a])
                plsc.store_scatter(hist, [a], pos + 1)
                plsc.store_scatter(ubuf, [lanes, jnp.full((BH,), i,
                                                          jnp.int32)],
                                   lanes * S + pos)
                return carry

            lax.fori_loop(0, _SEG, p3u, 0)
            pltpu.sync_copy(ubuf, uidx_hbm.at[:, pl.ds(tau * _SEG, _SEG)])

    return sort_k(bk_flat)


_NW = 32
_RPW = BH * S // _NW
_GCH = _RPW // 128


def _sc_gather_qkv(qk2, v2, sidx2):
    mesh = plsc.VectorSubcoreMesh(core_axis_name="c", subcore_axis_name="s")

    @functools.partial(
        pl.kernel,
        mesh=mesh,
        out_type=[jax.ShapeDtypeStruct((BH * S, DH), jnp.float32),
                  jax.ShapeDtypeStruct((BH * S, DH), jnp.float32)],
        scratch_types=[
            pltpu.VMEM((_GCH, 128), jnp.int32),
            pltpu.VMEM((128, DH), jnp.float32),
            pltpu.VMEM((128, DH), jnp.float32),
            pltpu.SemaphoreType.DMA,
            pltpu.SemaphoreType.DMA,
        ],
    )
    def gather_k(qk_hbm, v_hbm, idx_hbm, qks_hbm, vs_hbm,
                 idx_v, qbuf, vbuf, sem1, sem2):
        cid = lax.axis_index("c")
        sid = lax.axis_index("s")
        wid = sid * 2 + cid
        pltpu.sync_copy(idx_hbm.at[pl.ds(wid * _GCH, _GCH)], idx_v)
        for ch in range(_GCH):
            cp1 = pltpu.async_copy(qk_hbm.at[idx_v.at[ch]], qbuf, sem1)
            cp2 = pltpu.async_copy(v_hbm.at[idx_v.at[ch]], vbuf, sem2)
            cp1.wait()
            cp2.wait()
            row0 = wid * _RPW + ch * 128
            pltpu.sync_copy(qbuf, qks_hbm.at[pl.ds(row0, 128)])
            pltpu.sync_copy(vbuf, vs_hbm.at[pl.ds(row0, 128)])

    return gather_k(qk2, v2, sidx2)


def _sc_unsort_o(o_s2, uidx2):
    mesh = plsc.VectorSubcoreMesh(core_axis_name="c", subcore_axis_name="s")

    @functools.partial(
        pl.kernel,
        mesh=mesh,
        out_type=jax.ShapeDtypeStruct((BH * S, DH), jnp.float32),
        scratch_types=[
            pltpu.VMEM((_GCH, 128), jnp.int32),
            pltpu.VMEM((128, DH), jnp.float32),
            pltpu.SemaphoreType.DMA,
        ],
    )
    def unsort_k(os_hbm, idx_hbm, ot_hbm, idx_v, buf, sem):
        cid = lax.axis_index("c")
        sid = lax.axis_index("s")
        wid = sid * 2 + cid
        pltpu.sync_copy(idx_hbm.at[pl.ds(wid * _GCH, _GCH)], idx_v)
        for ch in range(_GCH):
            pltpu.async_copy(os_hbm.at[idx_v.at[ch]], buf, sem).wait()
            pltpu.sync_copy(buf, ot_hbm.at[pl.ds(wid * _RPW + ch * 128,
                                                 128)])

    return unsort_k(o_s2, uidx2)


def _pipeline(x, rot, Wqk, Wv, Wo, ln1_g, ln1_b, W1, b1, W2, b2,
              ln2_g, ln2_b, Wout, bout, interpret=False):
    rotc = jnp.concatenate([rot, -rot], axis=-1)
    rot_bd = jnp.zeros((D, H * NBK), jnp.float32)
    for hh in range(H):
        rot_bd = rot_bd.at[hh * DH:(hh + 1) * DH,
                           hh * NBK:(hh + 1) * NBK].set(rotc[hh])

    bf = jnp.bfloat16
    Wv, Wo, W1, W2, Wout = (w.astype(bf) for w in (Wv, Wo, W1, W2, Wout))

    qk, v, bk_bsh = _stage1(x, ln1_g, ln1_b, Wqk, Wv, rot_bd,
                            interpret=interpret)

    bk_flat = bk_bsh.transpose(1, 0, 2).reshape(S * BH)
    s_flat, u_flat = _sc_sort(bk_flat)

    qk2 = qk.reshape(BH * S, DH)
    v2 = v.reshape(BH * S, DH)
    qks, vs = _sc_gather_qkv(qk2, v2, s_flat.reshape(-1, 128))

    ts = s_flat.reshape(BH, 1, S)
    o_s = _attention(qks.reshape(BH, S, DH), vs.reshape(BH, S, DH), ts,
                     interpret=interpret)

    o_t = _sc_unsort_o(o_s.reshape(BH * S, DH),
                       u_flat.reshape(BH * S).reshape(-1, 128)
                       ).reshape(B, H, S, DH)

    return _stage6(x, o_t, Wo, ln2_g, ln2_b, W1, b1, W2, b2, Wout, bout,
                   interpret=interpret)


def kernel(x, rot, Wqk, Wv, Wo, ln1_g, ln1_b, W1, b1, W2, b2,
           ln2_g, ln2_b, Wout, bout):
    return _pipeline(x, rot, Wqk, Wv, Wo, ln1_g, ln1_b, W1, b1, W2, b2,
                     ln2_g, ln2_b, Wout, bout)
